# baseline (speedup 1.0000x reference)
import jax
import jax.numpy as jnp
from jax import lax
from jax.experimental import pallas as pl
from jax.experimental.pallas import tpu as pltpu
from jax.experimental.pallas import tpu_sc as plsc

D = 1024
ROWS = 512
NBUF = 8
SPLIT = 2


def _gather_body(ids_hbm, g_hbm, b_hbm, go_hbm, bo_hbm, ids_v, gv, bv, sem):
    wid = lax.axis_index("s") * 2 + lax.axis_index("c")

    @pl.when(wid == 0)
    def _():
        pltpu.sync_copy(ids_hbm, ids_v)
        pltpu.async_copy(g_hbm.at[ids_v], gv, sem).wait()
        pltpu.async_copy(b_hbm.at[ids_v], bv, sem).wait()
        pltpu.sync_copy(gv, go_hbm)
        pltpu.sync_copy(bv, bo_hbm)


def _sc_gather(ids, gamma, beta):
    n, d = gamma.shape
    mesh = plsc.VectorSubcoreMesh(core_axis_name="c", subcore_axis_name="s")
    return pl.kernel(
        _gather_body,
        out_type=(
            jax.ShapeDtypeStruct((n, d), gamma.dtype),
            jax.ShapeDtypeStruct((n, d), beta.dtype),
        ),
        mesh=mesh,
        scratch_types=[
            pltpu.VMEM((n,), jnp.int32),
            pltpu.VMEM((n, d), jnp.float32),
            pltpu.VMEM((n, d), jnp.float32),
            pltpu.SemaphoreType.DMA,
        ],
    )(ids, gamma, beta)


def _ring(x_hbm, o_hbm, buf, in_sems, out_sems, b_lo, b_hi, row_fn):
    _, S, _ = x_hbm.shape
    per_batch = S // ROWS
    n_chunks = (b_hi - b_lo) * per_batch

    def start_in(c, bi):
        b = b_lo + c // per_batch
        cp = pltpu.make_async_copy(
            x_hbm.at[b, pl.ds((c % per_batch) * ROWS, ROWS), :],
            buf.at[bi], in_sems.at[bi])
        cp.start()
        return cp

    def start_out(c, bi):
        b = b_lo + c // per_batch
        cp = pltpu.make_async_copy(
            buf.at[bi],
            o_hbm.at[b, pl.ds((c % per_batch) * ROWS, ROWS), :],
            out_sems.at[bi])
        cp.start()
        return cp

    in_cp = [None] * NBUF
    out_cp = [None] * NBUF
    for c in range(min(NBUF - 1, n_chunks)):
        in_cp[c] = start_in(c, c)
    for c in range(n_chunks):
        bi = c % NBUF
        batch = b_lo + c // per_batch
        in_cp[bi].wait()
        g_row, b_row = row_fn(batch)
        buf[bi] = g_row * buf[bi] + b_row
        out_cp[bi] = start_out(c, bi)
        nxt = c + NBUF - 1
        if nxt < n_chunks:
            nbi = nxt % NBUF
            if out_cp[nbi] is not None:
                out_cp[nbi].wait()
            in_cp[nbi] = start_in(nxt, nbi)
    for cp in out_cp:
        if cp is not None:
            cp.wait()


def _film1_body(ids_ref, x_hbm, g_ref, b_ref, o_hbm, buf, in_sems, out_sems):
    n = g_ref.shape[0]

    def row_fn(batch):
        idx = ids_ref[batch]
        mask = lax.broadcasted_iota(jnp.int32, (n, 1), 0) == idx
        g_row = jnp.sum(jnp.where(mask, g_ref[...], 0.0), axis=0)
        b_row = jnp.sum(jnp.where(mask, b_ref[...], 0.0), axis=0)
        return g_row, b_row

    _ring(x_hbm, o_hbm, buf, in_sems, out_sems, 0, SPLIT, row_fn)


def _film2_body(x_hbm, g_ref, b_ref, prev_hbm, o_hbm, buf, in_sems, out_sems):
    B = x_hbm.shape[0]
    _ring(x_hbm, o_hbm, buf, in_sems, out_sems, SPLIT, B,
          lambda batch: (g_ref[batch], b_ref[batch]))


@jax.jit
def _film(x, ids, gamma, beta):
    B, S, Dm = x.shape
    g_rows, b_rows = _sc_gather(ids, gamma, beta)
    out_shape = jax.ShapeDtypeStruct((B, S, Dm), x.dtype)
    scratch = [
        pltpu.VMEM((NBUF, ROWS, Dm), jnp.float32),
        pltpu.SemaphoreType.DMA((NBUF,)),
        pltpu.SemaphoreType.DMA((NBUF,)),
    ]
    out1 = pl.pallas_call(
        _film1_body,
        in_specs=[
            pl.BlockSpec(memory_space=pltpu.MemorySpace.SMEM),
            pl.BlockSpec(memory_space=pl.ANY),
            pl.BlockSpec(memory_space=pltpu.MemorySpace.VMEM),
            pl.BlockSpec(memory_space=pltpu.MemorySpace.VMEM),
        ],
        out_specs=pl.BlockSpec(memory_space=pl.ANY),
        out_shape=out_shape,
        scratch_shapes=scratch,
    )(ids, x, gamma, beta)
    return pl.pallas_call(
        _film2_body,
        in_specs=[
            pl.BlockSpec(memory_space=pl.ANY),
            pl.BlockSpec(memory_space=pltpu.MemorySpace.VMEM),
            pl.BlockSpec(memory_space=pltpu.MemorySpace.VMEM),
            pl.BlockSpec(memory_space=pl.ANY),
        ],
        out_specs=pl.BlockSpec(memory_space=pl.ANY),
        out_shape=out_shape,
        input_output_aliases={3: 0},
        scratch_shapes=scratch,
    )(x, g_rows, b_rows, out1)


def kernel(x, condition_ids, gamma, beta):
    return _film(x, condition_ids.astype(jnp.int32), gamma, beta)


# speedup vs baseline: 1.0027x; 1.0027x over previous
import jax
import jax.numpy as jnp
from jax import lax
from jax.experimental import pallas as pl
from jax.experimental.pallas import tpu as pltpu
from jax.experimental.pallas import tpu_sc as plsc

D = 1024
ROWS = 512
NBUF = 8
SPLIT = 3


def _gather_body(ids_hbm, g_hbm, b_hbm, go_hbm, bo_hbm, ids_v, gv, bv, sem):
    wid = lax.axis_index("s") * 2 + lax.axis_index("c")

    @pl.when(wid == 0)
    def _():
        pltpu.sync_copy(ids_hbm, ids_v)
        pltpu.async_copy(g_hbm.at[ids_v], gv, sem).wait()
        pltpu.async_copy(b_hbm.at[ids_v], bv, sem).wait()
        pltpu.sync_copy(gv, go_hbm)
        pltpu.sync_copy(bv, bo_hbm)


def _sc_gather(ids, gamma, beta):
    n, d = gamma.shape
    mesh = plsc.VectorSubcoreMesh(core_axis_name="c", subcore_axis_name="s")
    return pl.kernel(
        _gather_body,
        out_type=(
            jax.ShapeDtypeStruct((n, d), gamma.dtype),
            jax.ShapeDtypeStruct((n, d), beta.dtype),
        ),
        mesh=mesh,
        scratch_types=[
            pltpu.VMEM((n,), jnp.int32),
            pltpu.VMEM((n, d), jnp.float32),
            pltpu.VMEM((n, d), jnp.float32),
            pltpu.SemaphoreType.DMA,
        ],
    )(ids, gamma, beta)


def _ring(x_hbm, o_hbm, buf, in_sems, out_sems, b_lo, b_hi, row_fn):
    _, S, _ = x_hbm.shape
    per_batch = S // ROWS
    n_chunks = (b_hi - b_lo) * per_batch

    def start_in(c, bi):
        b = b_lo + c // per_batch
        cp = pltpu.make_async_copy(
            x_hbm.at[b, pl.ds((c % per_batch) * ROWS, ROWS), :],
            buf.at[bi], in_sems.at[bi])
        cp.start()
        return cp

    def start_out(c, bi):
        b = b_lo + c // per_batch
        cp = pltpu.make_async_copy(
            buf.at[bi],
            o_hbm.at[b, pl.ds((c % per_batch) * ROWS, ROWS), :],
            out_sems.at[bi])
        cp.start()
        return cp

    in_cp = [None] * NBUF
    out_cp = [None] * NBUF
    for c in range(min(NBUF - 1, n_chunks)):
        in_cp[c] = start_in(c, c)
    for c in range(n_chunks):
        bi = c % NBUF
        batch = b_lo + c // per_batch
        in_cp[bi].wait()
        g_row, b_row = row_fn(batch)
        buf[bi] = g_row * buf[bi] + b_row
        out_cp[bi] = start_out(c, bi)
        nxt = c + NBUF - 1
        if nxt < n_chunks:
            nbi = nxt % NBUF
            if out_cp[nbi] is not None:
                out_cp[nbi].wait()
            in_cp[nbi] = start_in(nxt, nbi)
    for cp in out_cp:
        if cp is not None:
            cp.wait()


def _film1_body(ids_ref, x_hbm, g_ref, b_ref, o_hbm, buf, in_sems, out_sems):
    n = g_ref.shape[0]

    def row_fn(batch):
        idx = ids_ref[batch]
        mask = lax.broadcasted_iota(jnp.int32, (n, 1), 0) == idx
        g_row = jnp.sum(jnp.where(mask, g_ref[...], 0.0), axis=0)
        b_row = jnp.sum(jnp.where(mask, b_ref[...], 0.0), axis=0)
        return g_row, b_row

    _ring(x_hbm, o_hbm, buf, in_sems, out_sems, 0, SPLIT, row_fn)


def _film2_body(x_hbm, g_ref, b_ref, prev_hbm, o_hbm, buf, in_sems, out_sems):
    B = x_hbm.shape[0]
    _ring(x_hbm, o_hbm, buf, in_sems, out_sems, SPLIT, B,
          lambda batch: (g_ref[batch], b_ref[batch]))


@jax.jit
def _film(x, ids, gamma, beta):
    B, S, Dm = x.shape
    g_rows, b_rows = _sc_gather(ids, gamma, beta)
    out_shape = jax.ShapeDtypeStruct((B, S, Dm), x.dtype)
    scratch = [
        pltpu.VMEM((NBUF, ROWS, Dm), jnp.float32),
        pltpu.SemaphoreType.DMA((NBUF,)),
        pltpu.SemaphoreType.DMA((NBUF,)),
    ]
    out1 = pl.pallas_call(
        _film1_body,
        in_specs=[
            pl.BlockSpec(memory_space=pltpu.MemorySpace.SMEM),
            pl.BlockSpec(memory_space=pl.ANY),
            pl.BlockSpec(memory_space=pltpu.MemorySpace.VMEM),
            pl.BlockSpec(memory_space=pltpu.MemorySpace.VMEM),
        ],
        out_specs=pl.BlockSpec(memory_space=pl.ANY),
        out_shape=out_shape,
        scratch_shapes=scratch,
    )(ids, x, gamma, beta)
    return pl.pallas_call(
        _film2_body,
        in_specs=[
            pl.BlockSpec(memory_space=pl.ANY),
            pl.BlockSpec(memory_space=pltpu.MemorySpace.VMEM),
            pl.BlockSpec(memory_space=pltpu.MemorySpace.VMEM),
            pl.BlockSpec(memory_space=pl.ANY),
        ],
        out_specs=pl.BlockSpec(memory_space=pl.ANY),
        out_shape=out_shape,
        input_output_aliases={3: 0},
        scratch_shapes=scratch,
    )(x, g_rows, b_rows, out1)


def kernel(x, condition_ids, gamma, beta):
    return _film(x, condition_ids.astype(jnp.int32), gamma, beta)


# speedup vs baseline: 1.0037x; 1.0010x over previous
import jax
import jax.numpy as jnp
from jax import lax
from jax.experimental import pallas as pl
from jax.experimental.pallas import tpu as pltpu
from jax.experimental.pallas import tpu_sc as plsc

D = 1024
ROWS = 512
NBUF = 8
SPLIT = 3


def _gather_body(ids_hbm, g_hbm, b_hbm, go_hbm, bo_hbm, ids_s, gsem, bsem):
    n = g_hbm.shape[0]

    @pl.when(lax.axis_index("c") == 0)
    def _():
        pltpu.sync_copy(ids_hbm, ids_s)
        gcp = [None] * n
        bcp = [None] * n
        for i in range(n):
            idx = ids_s[i]
            gcp[i] = pltpu.async_copy(g_hbm.at[idx], go_hbm.at[i], gsem)
            bcp[i] = pltpu.async_copy(b_hbm.at[idx], bo_hbm.at[i], bsem)
        for i in range(n):
            gcp[i].wait()
            bcp[i].wait()


def _sc_gather(ids, gamma, beta):
    n, d = gamma.shape
    mesh = plsc.ScalarSubcoreMesh(axis_name="c", num_cores=2)
    return pl.kernel(
        _gather_body,
        out_type=(
            jax.ShapeDtypeStruct((n, d), gamma.dtype),
            jax.ShapeDtypeStruct((n, d), beta.dtype),
        ),
        mesh=mesh,
        scratch_types=[
            pltpu.SMEM((n,), jnp.int32),
            pltpu.SemaphoreType.DMA,
            pltpu.SemaphoreType.DMA,
        ],
    )(ids, gamma, beta)


def _ring(x_hbm, o_hbm, buf, in_sems, out_sems, b_lo, b_hi, row_fn):
    _, S, _ = x_hbm.shape
    per_batch = S // ROWS
    n_chunks = (b_hi - b_lo) * per_batch

    def start_in(c, bi):
        b = b_lo + c // per_batch
        cp = pltpu.make_async_copy(
            x_hbm.at[b, pl.ds((c % per_batch) * ROWS, ROWS), :],
            buf.at[bi], in_sems.at[bi])
        cp.start()
        return cp

    def start_out(c, bi):
        b = b_lo + c // per_batch
        cp = pltpu.make_async_copy(
            buf.at[bi],
            o_hbm.at[b, pl.ds((c % per_batch) * ROWS, ROWS), :],
            out_sems.at[bi])
        cp.start()
        return cp

    in_cp = [None] * NBUF
    out_cp = [None] * NBUF
    for c in range(min(NBUF - 1, n_chunks)):
        in_cp[c] = start_in(c, c)
    for c in range(n_chunks):
        bi = c % NBUF
        batch = b_lo + c // per_batch
        in_cp[bi].wait()
        g_row, b_row = row_fn(batch)
        buf[bi] = g_row * buf[bi] + b_row
        out_cp[bi] = start_out(c, bi)
        nxt = c + NBUF - 1
        if nxt < n_chunks:
            nbi = nxt % NBUF
            if out_cp[nbi] is not None:
                out_cp[nbi].wait()
            in_cp[nbi] = start_in(nxt, nbi)
    for cp in out_cp:
        if cp is not None:
            cp.wait()


def _film1_body(ids_ref, x_hbm, g_ref, b_ref, o_hbm, buf, in_sems, out_sems):
    n = g_ref.shape[0]

    def row_fn(batch):
        idx = ids_ref[batch]
        mask = lax.broadcasted_iota(jnp.int32, (n, 1), 0) == idx
        g_row = jnp.sum(jnp.where(mask, g_ref[...], 0.0), axis=0)
        b_row = jnp.sum(jnp.where(mask, b_ref[...], 0.0), axis=0)
        return g_row, b_row

    _ring(x_hbm, o_hbm, buf, in_sems, out_sems, 0, SPLIT, row_fn)


def _film2_body(x_hbm, g_ref, b_ref, prev_hbm, o_hbm, buf, in_sems, out_sems):
    B = x_hbm.shape[0]
    _ring(x_hbm, o_hbm, buf, in_sems, out_sems, SPLIT, B,
          lambda batch: (g_ref[batch], b_ref[batch]))


@jax.jit
def _film(x, ids, gamma, beta):
    B, S, Dm = x.shape
    g_rows, b_rows = _sc_gather(ids, gamma, beta)
    out_shape = jax.ShapeDtypeStruct((B, S, Dm), x.dtype)
    scratch = [
        pltpu.VMEM((NBUF, ROWS, Dm), jnp.float32),
        pltpu.SemaphoreType.DMA((NBUF,)),
        pltpu.SemaphoreType.DMA((NBUF,)),
    ]
    out1 = pl.pallas_call(
        _film1_body,
        in_specs=[
            pl.BlockSpec(memory_space=pltpu.MemorySpace.SMEM),
            pl.BlockSpec(memory_space=pl.ANY),
            pl.BlockSpec(memory_space=pltpu.MemorySpace.VMEM),
            pl.BlockSpec(memory_space=pltpu.MemorySpace.VMEM),
        ],
        out_specs=pl.BlockSpec(memory_space=pl.ANY),
        out_shape=out_shape,
        scratch_shapes=scratch,
    )(ids, x, gamma, beta)
    return pl.pallas_call(
        _film2_body,
        in_specs=[
            pl.BlockSpec(memory_space=pl.ANY),
            pl.BlockSpec(memory_space=pltpu.MemorySpace.VMEM),
            pl.BlockSpec(memory_space=pltpu.MemorySpace.VMEM),
            pl.BlockSpec(memory_space=pl.ANY),
        ],
        out_specs=pl.BlockSpec(memory_space=pl.ANY),
        out_shape=out_shape,
        input_output_aliases={3: 0},
        scratch_shapes=scratch,
    )(x, g_rows, b_rows, out1)


def kernel(x, condition_ids, gamma, beta):
    return _film(x, condition_ids.astype(jnp.int32), gamma, beta)
